# baseline (speedup 1.0000x reference)
import functools

import jax
import jax.numpy as jnp
from jax import lax
from jax.experimental import pallas as pl
from jax.experimental.pallas import tpu as pltpu
from jax.experimental.pallas import tpu_sc as plsc

NC = 2
NS = 16
NW = NC * NS
IDXW = 128
GROUP = 256
NBUF = 4


@functools.partial(jax.jit, static_argnums=(2, 3))
def _emb_lookup(ids2d, table, n_rows, dim):
    rows_per_w = n_rows // NW
    n_g = rows_per_w // GROUP
    sub = GROUP // IDXW
    n_i = rows_per_w // IDXW
    mesh = plsc.VectorSubcoreMesh(core_axis_name="c", subcore_axis_name="s")

    @functools.partial(
        pl.kernel,
        out_type=jax.ShapeDtypeStruct((n_rows, dim), jnp.float32),
        mesh=mesh,
        scratch_types=[
            pltpu.VMEM((n_i, IDXW), jnp.int32),
            [pltpu.VMEM((GROUP, dim), jnp.float32) for _ in range(NBUF)],
            [pltpu.SemaphoreType.DMA for _ in range(NBUF)],
            [pltpu.SemaphoreType.DMA for _ in range(NBUF)],
        ],
        compiler_params=pltpu.CompilerParams(use_tc_tiling_on_sc=False),
    )
    def k(ids_hbm, table_hbm, out_hbm, idx_v, bufs, gsems, wsems):
        wid = lax.axis_index("s") * NC + lax.axis_index("c")
        base_row = wid * rows_per_w

        pltpu.sync_copy(ids_hbm.at[pl.ds(wid * n_i, n_i)], idx_v)

        def start_gather(g, b):
            for q in range(sub):
                pltpu.async_copy(
                    table_hbm.at[idx_v.at[g * sub + q]],
                    bufs[b].at[pl.ds(q * IDXW, IDXW)],
                    gsems[b],
                )

        def wait_gather(g, b):
            for q in range(sub):
                pltpu.make_async_copy(
                    table_hbm.at[idx_v.at[g * sub + q]],
                    bufs[b].at[pl.ds(q * IDXW, IDXW)],
                    gsems[b],
                ).wait()

        def start_write(g, b):
            pltpu.async_copy(
                bufs[b], out_hbm.at[pl.ds(base_row + g * GROUP, GROUP)],
                wsems[b],
            )

        def wait_write(g, b):
            pltpu.make_async_copy(
                bufs[b], out_hbm.at[pl.ds(base_row + g * GROUP, GROUP)],
                wsems[b],
            ).wait()

        for b in range(NBUF):
            start_gather(b, b)

        def outer(go, _):
            for b in range(NBUF):
                g = go * NBUF + b
                wait_gather(g, b)
                start_write(g, b)
            for b in range(NBUF):
                g = go * NBUF + b
                wait_write(g, b)
                start_gather(g + NBUF, b)
            return _

        lax.fori_loop(0, n_g // NBUF - 1, outer, 0, unroll=False)

        for b in range(NBUF):
            g = n_g - NBUF + b
            wait_gather(g, b)
            start_write(g, b)
        for b in range(NBUF):
            g = n_g - NBUF + b
            wait_write(g, b)

    return k(ids2d, table)


def kernel(input_ids, weight):
    batch, seq = input_ids.shape
    vocab, dim = weight.shape
    n_rows = batch * seq
    assert n_rows % (NW * GROUP * NBUF) == 0 and dim % 16 == 0
    ids2d = input_ids.astype(jnp.int32).reshape(n_rows // IDXW, IDXW)
    out = _emb_lookup(ids2d, weight, n_rows, dim)
    return out.reshape(batch, seq, dim)


# speedup vs baseline: 1.1380x; 1.1380x over previous
import functools

import jax
import jax.numpy as jnp
from jax import lax
from jax.experimental import pallas as pl
from jax.experimental.pallas import tpu as pltpu
from jax.experimental.pallas import tpu_sc as plsc

NC, NS = 2, 16
NW = NC * NS

_TC_PARAMS = pltpu.CompilerParams(
    use_tc_tiling_on_sc=True, needs_layout_passes=False
)


def _mesh():
    return plsc.VectorSubcoreMesh(core_axis_name="c", subcore_axis_name="s")


U0 = 21


@functools.partial(
    pl.kernel,
    out_type=jax.ShapeDtypeStruct((500000, 128), jnp.float32),
    mesh=_mesh(),
    scratch_types=[
        [pltpu.VMEM((U0, 16, 64), jnp.float32) for _ in range(2)],
        [pltpu.VMEM((U0 * 8, 128), jnp.float32) for _ in range(2)],
        [pltpu.SemaphoreType.DMA for _ in range(2)],
        [pltpu.SemaphoreType.DMA for _ in range(2)],
    ],
    compiler_params=_TC_PARAMS,
)
def _depad(w3, out, bufs, obufs, isems, osems):
    wid = lax.axis_index("s") * NC + lax.axis_index("c")
    base = 1953 * wid

    def start_in(u0, b, n):
        pltpu.async_copy(w3.at[pl.ds(u0, n)], bufs[b].at[pl.ds(0, n)], isems[b])

    def wait_in(u0, b, n):
        pltpu.make_async_copy(
            w3.at[pl.ds(u0, n)], bufs[b].at[pl.ds(0, n)], isems[b]
        ).wait()

    def repack(b, n):
        def ubody(u, _):
            for rr in range(16):
                for k in range(4):
                    v = bufs[b][u, rr, pl.ds(k * 16, 16)]
                    obufs[b][u * 8 + rr // 2, pl.ds((rr % 2) * 64 + k * 16, 16)] = v
            return _

        lax.fori_loop(0, n, ubody, 0, unroll=3)

    def start_out(u0, b, n):
        r0 = u0 * 8
        pltpu.async_copy(
            obufs[b].at[pl.ds(0, n * 8)], out.at[pl.ds(r0, n * 8)], osems[b]
        )

    def wait_out(u0, b, n):
        r0 = u0 * 8
        pltpu.make_async_copy(
            obufs[b].at[pl.ds(0, n * 8)], out.at[pl.ds(r0, n * 8)], osems[b]
        ).wait()

    start_in(base, 0, U0)
    start_in(base + U0, 1, U0)

    def body(i, _):
        u0 = base + i * U0
        b = lax.rem(i, 2)

        @pl.when(b == 0)
        def _b0():
            wait_in(u0, 0, U0)

            @pl.when(i >= 2)
            def _d():
                wait_out(u0 - 2 * U0, 0, U0)

            repack(0, U0)
            start_out(u0, 0, U0)
            start_in(u0 + 2 * U0, 0, U0)

        @pl.when(b == 1)
        def _b1():
            wait_in(u0, 1, U0)

            @pl.when(i >= 2)
            def _d():
                wait_out(u0 - 2 * U0, 1, U0)

            repack(1, U0)
            start_out(u0, 1, U0)
            start_in(u0 + 2 * U0, 1, U0)

        return _

    lax.fori_loop(0, 91, body, 0, unroll=False)
    for i, b in ((91, 1), (92, 0)):
        u0 = base + i * U0
        wait_in(u0, b, U0)
        wait_out(u0 - 2 * U0, b, U0)
        repack(b, U0)
        start_out(u0, b, U0)
    for i, b in ((91, 1), (92, 0)):
        wait_out(base + i * U0, b, U0)

    @pl.when(wid < 4)
    def _tail():
        u0 = 1953 * 32 + wid
        start_in(u0, 0, 1)
        wait_in(u0, 0, 1)
        repack(0, 1)
        start_out(u0, 0, 1)
        wait_out(u0, 0, 1)


RPW = 25600
NS_TOT = 200


@functools.partial(
    pl.kernel,
    out_type=jax.ShapeDtypeStruct((200, 64, 4096), jnp.float32),
    mesh=_mesh(),
    scratch_types=[
        pltpu.VMEM((RPW,), jnp.int32),
        [pltpu.VMEM((128,), jnp.int32) for _ in range(4)],
        [pltpu.VMEM((128, 128), jnp.float32) for _ in range(4)],
        [pltpu.VMEM((64, 128), jnp.float32) for _ in range(4)],
        [pltpu.SemaphoreType.DMA for _ in range(4)],
        [pltpu.SemaphoreType.DMA for _ in range(4)],
    ],
    compiler_params=_TC_PARAMS,
)
def _gather_t(ids1d, tpairs, out, idsv, pbufs, gbufs, obufs, gsems, osems):
    wid = lax.axis_index("s") * NC + lax.axis_index("c")
    b0 = wid * 128

    pltpu.sync_copy(ids1d.at[pl.ds(wid * RPW, RPW)], idsv)

    lanes = lax.iota(jnp.int32, 16)
    jbase = [lanes * 200 + (q * 16 * 200) for q in range(8)]

    def extract(s, b):
        hb = []
        for q in range(8):
            ids_q = plsc.load_gather(idsv, [jbase[q] + s])
            pbufs[b][pl.ds(q * 16, 16)] = lax.shift_right_logical(ids_q, 1)
            hb.append(lax.shift_left(jnp.bitwise_and(ids_q, 1), 6))
        return hb

    def start_gather(b):
        pltpu.async_copy(tpairs.at[pbufs[b]], gbufs[b], gsems[b])

    def wait_gather(b):
        pltpu.make_async_copy(tpairs.at[pbufs[b]], gbufs[b], gsems[b]).wait()

    def start_out(s, b):
        pltpu.async_copy(obufs[b], out.at[s, :, pl.ds(b0, 128)], osems[b])

    def wait_out(s, b):
        pltpu.make_async_copy(obufs[b], out.at[s, :, pl.ds(b0, 128)], osems[b]).wait()

    def transpose(hb, b):
        rows = [lanes + q * 16 for q in range(8)]

        def tbody(t, _):
            tm = jnp.bitwise_and(t, 15)
            dvec = (t - tm) + jnp.bitwise_and(lanes + tm, 15)
            for q in range(8):
                v = plsc.load_gather(gbufs[b], [rows[q], hb[q] + dvec])
                plsc.store_scatter(obufs[b], [dvec, rows[q]], v)
            return _

        lax.fori_loop(0, 64, tbody, 0, unroll=4)

    hbs = [extract(b, b) for b in range(4)]
    for b in range(4):
        start_gather(b)

    def sbody(go, hbs4):
        s0 = 4 * go
        out_hbs = []
        for b in range(4):
            s = s0 + b
            wait_gather(b)

            @pl.when(s0 >= 4)
            def _d():
                wait_out(s - 4, b)

            transpose(hbs4[b], b)
            start_out(s, b)
            hb_next = extract(jnp.minimum(s + 4, 199), b)
            start_gather(b)
            out_hbs.append(hb_next)
        return out_hbs

    hbs = lax.fori_loop(0, 50, sbody, hbs, unroll=False)
    for b in range(4):
        wait_gather(b)
        wait_out(196 + b, b)


def kernel(input_ids, weight):
    assert input_ids.shape == (4096, 200) and weight.shape == (1000000, 64)
    w3 = weight.reshape(62500, 16, 64)
    tpairs = _depad(w3)
    ids1d = input_ids.astype(jnp.int32).reshape(819200)
    out_t = _gather_t(ids1d, tpairs)
    return out_t.transpose(2, 0, 1)


# speedup vs baseline: 1.3033x; 1.1452x over previous
import functools

import jax
import jax.numpy as jnp
from jax import lax
from jax.experimental import pallas as pl
from jax.experimental.pallas import tpu as pltpu
from jax.experimental.pallas import tpu_sc as plsc

NC, NS = 2, 16
NW = NC * NS

_TC_PARAMS = pltpu.CompilerParams(
    use_tc_tiling_on_sc=True, needs_layout_passes=False
)


def _mesh():
    return plsc.VectorSubcoreMesh(core_axis_name="c", subcore_axis_name="s")


U0 = 21


@functools.partial(
    pl.kernel,
    out_type=jax.ShapeDtypeStruct((500000, 128), jnp.float32),
    mesh=_mesh(),
    scratch_types=[
        [pltpu.VMEM((U0, 16, 64), jnp.float32) for _ in range(2)],
        [pltpu.VMEM((U0 * 8, 128), jnp.float32) for _ in range(2)],
        [pltpu.SemaphoreType.DMA for _ in range(2)],
        [pltpu.SemaphoreType.DMA for _ in range(2)],
    ],
    compiler_params=_TC_PARAMS,
)
def _depad(w3, out, bufs, obufs, isems, osems):
    wid = lax.axis_index("s") * NC + lax.axis_index("c")
    base = 1953 * wid

    def start_in(u0, b, n):
        pltpu.async_copy(w3.at[pl.ds(u0, n)], bufs[b].at[pl.ds(0, n)], isems[b])

    def wait_in(u0, b, n):
        pltpu.make_async_copy(
            w3.at[pl.ds(u0, n)], bufs[b].at[pl.ds(0, n)], isems[b]
        ).wait()

    def repack(b, n):
        for u in range(n):
            for rr in range(16):
                for k in range(4):
                    v = bufs[b][u, rr, pl.ds(k * 16, 16)]
                    obufs[b][u * 8 + rr // 2, pl.ds((rr % 2) * 64 + k * 16, 16)] = v

    def start_out(u0, b, n):
        r0 = u0 * 8
        pltpu.async_copy(
            obufs[b].at[pl.ds(0, n * 8)], out.at[pl.ds(r0, n * 8)], osems[b]
        )

    def wait_out(u0, b, n):
        r0 = u0 * 8
        pltpu.make_async_copy(
            obufs[b].at[pl.ds(0, n * 8)], out.at[pl.ds(r0, n * 8)], osems[b]
        ).wait()

    start_in(base, 0, U0)
    start_in(base + U0, 1, U0)

    def body(i, _):
        u0 = base + i * U0
        b = lax.rem(i, 2)

        @pl.when(b == 0)
        def _b0():
            wait_in(u0, 0, U0)

            @pl.when(i >= 2)
            def _d():
                wait_out(u0 - 2 * U0, 0, U0)

            repack(0, U0)
            start_out(u0, 0, U0)
            start_in(u0 + 2 * U0, 0, U0)

        @pl.when(b == 1)
        def _b1():
            wait_in(u0, 1, U0)

            @pl.when(i >= 2)
            def _d():
                wait_out(u0 - 2 * U0, 1, U0)

            repack(1, U0)
            start_out(u0, 1, U0)
            start_in(u0 + 2 * U0, 1, U0)

        return _

    lax.fori_loop(0, 91, body, 0, unroll=False)
    for i, b in ((91, 1), (92, 0)):
        u0 = base + i * U0
        wait_in(u0, b, U0)
        wait_out(u0 - 2 * U0, b, U0)
        repack(b, U0)
        start_out(u0, b, U0)
    for i, b in ((91, 1), (92, 0)):
        wait_out(base + i * U0, b, U0)

    @pl.when(wid < 4)
    def _tail():
        u0 = 1953 * 32 + wid
        start_in(u0, 0, 1)
        wait_in(u0, 0, 1)
        repack(0, 1)
        start_out(u0, 0, 1)
        wait_out(u0, 0, 1)


RPW = 25600
NS_TOT = 200


@functools.partial(
    pl.kernel,
    out_type=jax.ShapeDtypeStruct((200, 64, 4096), jnp.float32),
    mesh=_mesh(),
    scratch_types=[
        pltpu.VMEM((RPW,), jnp.int32),
        [pltpu.VMEM((128,), jnp.int32) for _ in range(4)],
        [pltpu.VMEM((128, 128), jnp.float32) for _ in range(4)],
        [pltpu.VMEM((64, 128), jnp.float32) for _ in range(4)],
        [pltpu.SemaphoreType.DMA for _ in range(4)],
        [pltpu.SemaphoreType.DMA for _ in range(4)],
    ],
    compiler_params=_TC_PARAMS,
)
def _gather_t(ids1d, tpairs, out, idsv, pbufs, gbufs, obufs, gsems, osems):
    wid = lax.axis_index("s") * NC + lax.axis_index("c")
    b0 = wid * 128

    pltpu.sync_copy(ids1d.at[pl.ds(wid * RPW, RPW)], idsv)

    lanes = lax.iota(jnp.int32, 16)
    jbase = [lanes * 200 + (q * 16 * 200) for q in range(8)]

    def extract(s, b):
        hb = []
        for q in range(8):
            ids_q = plsc.load_gather(idsv, [jbase[q] + s])
            pbufs[b][pl.ds(q * 16, 16)] = lax.shift_right_logical(ids_q, 1)
            hb.append(lax.shift_left(jnp.bitwise_and(ids_q, 1), 6))
        return hb

    def start_gather(b):
        pltpu.async_copy(tpairs.at[pbufs[b]], gbufs[b], gsems[b])

    def wait_gather(b):
        pltpu.make_async_copy(tpairs.at[pbufs[b]], gbufs[b], gsems[b]).wait()

    def start_out(s, b):
        pltpu.async_copy(obufs[b], out.at[s, :, pl.ds(b0, 128)], osems[b])

    def wait_out(s, b):
        pltpu.make_async_copy(obufs[b], out.at[s, :, pl.ds(b0, 128)], osems[b]).wait()

    def transpose(hb, b):
        rows = [lanes + q * 16 for q in range(8)]

        def tbody(t, _):
            tm = jnp.bitwise_and(t, 15)
            dvec = (t - tm) + jnp.bitwise_and(lanes + tm, 15)
            for q in range(8):
                v = plsc.load_gather(gbufs[b], [rows[q], hb[q] + dvec])
                plsc.store_scatter(obufs[b], [dvec, rows[q]], v)
            return _

        lax.fori_loop(0, 64, tbody, 0, unroll=4)

    hbs = [extract(b, b) for b in range(4)]
    for b in range(4):
        start_gather(b)

    def sbody(go, hbs4):
        s0 = 4 * go
        out_hbs = []
        for b in range(4):
            s = s0 + b
            wait_gather(b)

            @pl.when(s0 >= 4)
            def _d():
                wait_out(s - 4, b)

            transpose(hbs4[b], b)
            start_out(s, b)
            hb_next = extract(jnp.minimum(s + 4, 199), b)
            start_gather(b)
            out_hbs.append(hb_next)
        return out_hbs

    hbs = lax.fori_loop(0, 50, sbody, hbs, unroll=False)
    for b in range(4):
        wait_gather(b)
        wait_out(196 + b, b)


def kernel(input_ids, weight):
    assert input_ids.shape == (4096, 200) and weight.shape == (1000000, 64)
    w3 = weight.reshape(62500, 16, 64)
    tpairs = _depad(w3)
    ids1d = input_ids.astype(jnp.int32).reshape(819200)
    out_t = _gather_t(ids1d, tpairs)
    return out_t.transpose(2, 0, 1)
